# baseline (speedup 1.0000x reference)
import jax
import jax.numpy as jnp
from jax import lax
from jax.experimental import pallas as pl
from jax.experimental.pallas import tpu as pltpu
from jax.experimental.pallas import tpu_sc as plsc

BATCH = 16384
SEQ = 50
DIM = 64
NUM_WORKERS = 32
B_PER_W = BATCH // NUM_WORKERS
NB = 4
ROWS = NB * SEQ
SLOTS = B_PER_W // NB
NBUF = 4
GROUPS = SLOTS // NBUF - 1


def _emb_body(table_hbm, idx_hbm, out_hbm, idx_v, rows_v, *sems):
    gsem = sems[:NBUF]
    wsem = sems[NBUF:]
    wid = lax.axis_index("s") * 2 + lax.axis_index("c")
    base = wid * B_PER_W
    pltpu.sync_copy(idx_hbm.at[pl.ds(base, B_PER_W)], idx_v)

    def gathers(slot, b):
        for j in range(NB):
            pltpu.async_copy(
                table_hbm.at[idx_v.at[slot * NB + j]],
                rows_v.at[b].at[pl.ds(j * SEQ, SEQ)], gsem[b])

    def gathers_wait(slot, b):
        for j in range(NB):
            pltpu.make_async_copy(
                table_hbm.at[idx_v.at[slot * NB + j]],
                rows_v.at[b].at[pl.ds(j * SEQ, SEQ)], gsem[b]).wait()

    def wb(slot, b):
        pltpu.async_copy(rows_v.at[b],
                         out_hbm.at[pl.ds((base + slot * NB) * SEQ, ROWS)],
                         wsem[b])

    def wb_wait(slot, b):
        pltpu.make_async_copy(rows_v.at[b],
                              out_hbm.at[pl.ds((base + slot * NB) * SEQ, ROWS)],
                              wsem[b]).wait()

    for b in range(NBUF):
        gathers(b, b)

    def group(g, carry):
        for b in range(NBUF):
            slot = g * NBUF + b
            gathers_wait(slot, b)
            wb(slot, b)
            wb_wait(slot, b)
            gathers(slot + NBUF, b)
        return carry

    lax.fori_loop(0, GROUPS, group, 0)

    for b in range(NBUF):
        slot = GROUPS * NBUF + b
        gathers_wait(slot, b)
        wb(slot, b)
    for b in range(NBUF):
        wb_wait(GROUPS * NBUF + b, b)


@jax.jit
def kernel(token_ids, indexing):
    mesh = plsc.VectorSubcoreMesh(core_axis_name="c", subcore_axis_name="s")
    out = pl.kernel(
        _emb_body,
        out_type=jax.ShapeDtypeStruct((BATCH * SEQ, DIM), jnp.float32),
        mesh=mesh,
        scratch_types=[
            pltpu.VMEM((B_PER_W, SEQ), jnp.int32),
            pltpu.VMEM((NBUF, ROWS, DIM), jnp.float32),
        ] + [pltpu.SemaphoreType.DMA] * (2 * NBUF),
        compiler_params=pltpu.CompilerParams(use_tc_tiling_on_sc=False),
    )(indexing, token_ids)
    return out.reshape(token_ids.shape + (DIM,))
